# baseline (speedup 1.0000x reference)
import jax
import jax.numpy as jnp
from jax import lax
from jax.experimental import pallas as pl

N = 8192
D = 16
KNN = 64
INV_SCALE = 2.0
RB = 256
NBLK = N // RB
MAX_FINITE_BITS = 0x7F7FFFFF


def _dist2(xb, xt, n2c, n2r):
    g = lax.dot_general(xb, xt, (((1,), (0,)), ((), ())),
                        preferred_element_type=jnp.float32)
    return jnp.maximum(n2c + n2r - 2.0 * g, 0.0)


def _norms_body(x_ref, n2_ref):
    xb = x_ref[...]
    n2_ref[...] = jnp.sum(xb * xb, axis=1, keepdims=True)


def _colsum(m):
    ones = jnp.ones((N, 1), jnp.float32)
    return lax.dot_general(m, ones, (((1,), (0,)), ((), ())),
                           preferred_element_type=jnp.float32)


def _select_body(x_ref, xt_ref, n2c_ref, n2r_ref, dens_ref, tau_ref):
    d2 = _dist2(x_ref[...], xt_ref[...], n2c_ref[...], n2r_ref[...])
    bits = lax.bitcast_convert_type(d2, jnp.int32)
    lo = jnp.zeros((RB, 1), jnp.int32)
    hi = jnp.full((RB, 1), MAX_FINITE_BITS, jnp.int32)

    def bs(_, lohi):
        lo, hi = lohi
        mid = lo + lax.div(hi - lo, 2)
        cnt = _colsum(jnp.where(bits <= mid, 1.0, 0.0))
        ge = cnt >= jnp.float32(KNN)
        return jnp.where(ge, lo, mid + 1), jnp.where(ge, mid, hi)

    _, tau_bits = lax.fori_loop(0, 31, bs, (lo, hi))
    tau2 = lax.bitcast_convert_type(tau_bits, jnp.float32)
    ltf = jnp.where(bits < tau_bits, 1.0, 0.0)
    c1 = _colsum(ltf)
    ed = jnp.exp(-jnp.sqrt(d2) * INV_SCALE)
    s1 = _colsum(ltf * ed)
    te = jnp.exp(-jnp.sqrt(tau2) * INV_SCALE)
    dens_ref[...] = s1 + (jnp.float32(KNN) - c1) * te
    tau_ref[...] = tau2


def _rank_body(densc_ref, densr_ref, rank_ref):
    i = pl.program_id(0)
    du = densc_ref[...]
    dw = densr_ref[...]
    iu = lax.broadcasted_iota(jnp.int32, (RB, 1), 0) + i * RB
    iw = lax.broadcasted_iota(jnp.int32, (1, N), 1)
    less = (dw < du) | ((dw == du) & (iw < iu))
    rank_ref[...] = _colsum(jnp.where(less, 1.0, 0.0)).astype(jnp.int32)


def _cc_body(x_ref, xt_ref, n2c_ref, n2r_ref, tauc_ref, taur_ref,
             rankc_ref, rankr_ref, pc_ref, pr_ref, pcolf_ref, pout_ref):
    d2 = _dist2(x_ref[...], xt_ref[...], n2c_ref[...], n2r_ref[...])
    ru = rankc_ref[...]
    rv = rankr_ref[...]
    pv = pr_ref[...]
    taueff = jnp.where(rv > ru, tauc_ref[...], taur_ref[...])
    edge = (d2 <= taueff) & (rv != ru)
    pnew = jnp.maximum(
        pc_ref[...],
        jnp.max(jnp.where(edge, pv, -1), axis=1, keepdims=True))
    pcolf = pcolf_ref[...]
    for _ in range(2):
        onehot = jnp.where(rv == pnew, 1.0, 0.0)
        pnew = lax.dot_general(onehot, pcolf, (((1,), (0,)), ((), ())),
                               preferred_element_type=jnp.float32
                               ).astype(jnp.int32)
    pout_ref[...] = pnew


def _label_body(pc_ref, pr_ref, rankr_ref, lab_ref):
    pu = pc_ref[...]
    pw = pr_ref[...]
    rw = rankr_ref[...]
    isroot = pw == rw
    lab_ref[...] = _colsum(jnp.where(isroot & (rw < pu), 1.0, 0.0)
                           ).astype(jnp.int32)


def kernel(x):
    x = x.astype(jnp.float32)
    xt = x.T

    f32 = jnp.float32
    i32 = jnp.int32
    sds = jax.ShapeDtypeStruct

    n2 = pl.pallas_call(
        _norms_body, out_shape=sds((N, 1), f32))(x)
    n2r = n2.reshape(1, N)

    spec_x = pl.BlockSpec((RB, D), lambda i: (i, 0))
    spec_xt = pl.BlockSpec((D, N), lambda i: (0, 0))
    spec_col = pl.BlockSpec((RB, 1), lambda i: (i, 0))
    spec_row = pl.BlockSpec((1, N), lambda i: (0, 0))

    dens, tau = pl.pallas_call(
        _select_body,
        grid=(NBLK,),
        in_specs=[spec_x, spec_xt, spec_col, spec_row],
        out_specs=[spec_col, spec_col],
        out_shape=[sds((N, 1), f32), sds((N, 1), f32)],
    )(x, xt, n2, n2r)
    densr = dens.reshape(1, N)
    taur = tau.reshape(1, N)

    rank = pl.pallas_call(
        _rank_body,
        grid=(NBLK,),
        in_specs=[spec_col, spec_row],
        out_specs=spec_col,
        out_shape=sds((N, 1), i32),
    )(dens, densr)
    rankr = rank.reshape(1, N)

    spec_colfull = pl.BlockSpec((N, 1), lambda i: (0, 0))
    cc_call = pl.pallas_call(
        _cc_body,
        grid=(NBLK,),
        in_specs=[spec_x, spec_xt, spec_col, spec_row, spec_col, spec_row,
                  spec_col, spec_row, spec_col, spec_row, spec_colfull],
        out_specs=spec_col,
        out_shape=sds((N, 1), i32),
    )

    def cond(carry):
        return carry[1]

    def body(carry):
        p, _ = carry
        pn = cc_call(x, xt, n2, n2r, tau, taur, rank, rankr,
                     p, p.reshape(1, N), p.astype(jnp.float32))
        return pn, jnp.any(pn != p)

    p, _ = lax.while_loop(cond, body, (rank, jnp.bool_(True)))

    labels = pl.pallas_call(
        _label_body,
        grid=(NBLK,),
        in_specs=[spec_col, spec_row, spec_row],
        out_specs=spec_col,
        out_shape=sds((N, 1), i32),
    )(p, p.reshape(1, N), rankr)

    return jnp.asarray(labels.reshape(N), jnp.int64)


# speedup vs baseline: 1.2339x; 1.2339x over previous
import jax
import jax.numpy as jnp
from jax import lax
from jax.experimental import pallas as pl

N = 8192
D = 16
KNN = 64
INV_SCALE = 2.0
RB = 256
NBLK = N // RB
MAX_FINITE_BITS = 0x7F7FFFFF


def _dist2(xb, xt, n2c, n2r):
    g = lax.dot_general(xb, xt, (((1,), (0,)), ((), ())),
                        preferred_element_type=jnp.float32)
    return jnp.maximum(n2c + n2r - 2.0 * g, 0.0)


def _norms_body(x_ref, n2_ref):
    xb = x_ref[...]
    n2_ref[...] = jnp.sum(xb * xb, axis=1, keepdims=True)


def _select_body(x_ref, xt_ref, n2c_ref, n2r_ref, dens_ref, tau_ref):
    d2 = _dist2(x_ref[...], xt_ref[...], n2c_ref[...], n2r_ref[...])
    bits = lax.bitcast_convert_type(d2, jnp.int32)
    lo = jnp.zeros((RB, 1), jnp.int32)
    hi = jnp.full((RB, 1), MAX_FINITE_BITS, jnp.int32)

    def bs(_, lohi):
        lo, hi = lohi
        mid = lo + lax.div(hi - lo, 2)
        cnt = jnp.sum((bits <= mid).astype(jnp.int32), axis=1, keepdims=True)
        ge = cnt >= KNN
        return jnp.where(ge, lo, mid + 1), jnp.where(ge, mid, hi)

    _, tau_bits = lax.fori_loop(0, 31, bs, (lo, hi))
    tau2 = lax.bitcast_convert_type(tau_bits, jnp.float32)
    lt = bits < tau_bits
    c1 = jnp.sum(lt.astype(jnp.float32), axis=1, keepdims=True)
    ed = jnp.exp(-jnp.sqrt(d2) * INV_SCALE)
    s1 = jnp.sum(jnp.where(lt, ed, 0.0), axis=1, keepdims=True)
    te = jnp.exp(-jnp.sqrt(tau2) * INV_SCALE)
    dens_ref[...] = s1 + (jnp.float32(KNN) - c1) * te
    tau_ref[...] = tau2


def _rank_body(densc_ref, densr_ref, rank_ref):
    i = pl.program_id(0)
    du = densc_ref[...]
    dw = densr_ref[...]
    iu = lax.broadcasted_iota(jnp.int32, (RB, 1), 0) + i * RB
    iw = lax.broadcasted_iota(jnp.int32, (1, N), 1)
    less = (dw < du) | ((dw == du) & (iw < iu))
    rank_ref[...] = jnp.sum(less.astype(jnp.int32), axis=1, keepdims=True)


def _cc_body(x_ref, xt_ref, n2c_ref, n2r_ref, tauc_ref, taur_ref,
             rankc_ref, rankr_ref, pc_ref, pr_ref, pout_ref):
    d2 = _dist2(x_ref[...], xt_ref[...], n2c_ref[...], n2r_ref[...])
    ru = rankc_ref[...]
    rv = rankr_ref[...]
    pv = pr_ref[...]
    taueff = jnp.where(rv > ru, tauc_ref[...],
                       jnp.where(rv < ru, taur_ref[...], -1.0))
    edge = d2 <= taueff
    pnew = jnp.maximum(
        pc_ref[...],
        jnp.max(jnp.where(edge, pv, -1), axis=1, keepdims=True))
    for _ in range(2):
        pnew = jnp.max(jnp.where(rv == pnew, pv, -1), axis=1, keepdims=True)
    pout_ref[...] = pnew


def _label_body(pc_ref, pr_ref, rankr_ref, lab_ref):
    pu = pc_ref[...]
    pw = pr_ref[...]
    rw = rankr_ref[...]
    isroot = pw == rw
    lab_ref[...] = jnp.sum((isroot & (rw < pu)).astype(jnp.int32),
                           axis=1, keepdims=True)


def kernel(x):
    x = x.astype(jnp.float32)
    xt = x.T

    f32 = jnp.float32
    i32 = jnp.int32
    sds = jax.ShapeDtypeStruct

    n2 = pl.pallas_call(
        _norms_body, out_shape=sds((N, 1), f32))(x)
    n2r = n2.reshape(1, N)

    spec_x = pl.BlockSpec((RB, D), lambda i: (i, 0))
    spec_xt = pl.BlockSpec((D, N), lambda i: (0, 0))
    spec_col = pl.BlockSpec((RB, 1), lambda i: (i, 0))
    spec_row = pl.BlockSpec((1, N), lambda i: (0, 0))

    dens, tau = pl.pallas_call(
        _select_body,
        grid=(NBLK,),
        in_specs=[spec_x, spec_xt, spec_col, spec_row],
        out_specs=[spec_col, spec_col],
        out_shape=[sds((N, 1), f32), sds((N, 1), f32)],
    )(x, xt, n2, n2r)
    densr = dens.reshape(1, N)
    taur = tau.reshape(1, N)

    rank = pl.pallas_call(
        _rank_body,
        grid=(NBLK,),
        in_specs=[spec_col, spec_row],
        out_specs=spec_col,
        out_shape=sds((N, 1), i32),
    )(dens, densr)
    rankr = rank.reshape(1, N)

    cc_call = pl.pallas_call(
        _cc_body,
        grid=(NBLK,),
        in_specs=[spec_x, spec_xt, spec_col, spec_row, spec_col, spec_row,
                  spec_col, spec_row, spec_col, spec_row],
        out_specs=spec_col,
        out_shape=sds((N, 1), i32),
    )

    def cond(carry):
        return carry[1]

    def body(carry):
        p, _ = carry
        pn = cc_call(x, xt, n2, n2r, tau, taur, rank, rankr,
                     p, p.reshape(1, N))
        return pn, jnp.any(pn != p)

    p, _ = lax.while_loop(cond, body, (rank, jnp.bool_(True)))

    labels = pl.pallas_call(
        _label_body,
        grid=(NBLK,),
        in_specs=[spec_col, spec_row, spec_row],
        out_specs=spec_col,
        out_shape=sds((N, 1), i32),
    )(p, p.reshape(1, N), rankr)

    return jnp.asarray(labels.reshape(N), jnp.int64)


# speedup vs baseline: 1.2732x; 1.0318x over previous
import jax
import jax.numpy as jnp
from jax import lax
from jax.experimental import pallas as pl

N = 8192
D = 16
KNN = 64
INV_SCALE = 2.0
RB = 512
NBLK = N // RB
MAX_FINITE_BITS = 0x7F7FFFFF


def _dist2(xb, xt, n2c, n2r):
    g = lax.dot_general(xb, xt, (((1,), (0,)), ((), ())),
                        preferred_element_type=jnp.float32)
    return jnp.maximum(n2c + n2r - 2.0 * g, 0.0)


def _norms_body(x_ref, n2_ref):
    xb = x_ref[...]
    n2_ref[...] = jnp.sum(xb * xb, axis=1, keepdims=True)


def _select_body(x_ref, xt_ref, n2c_ref, n2r_ref, dens_ref, tau_ref):
    d2 = _dist2(x_ref[...], xt_ref[...], n2c_ref[...], n2r_ref[...])
    bits = lax.bitcast_convert_type(d2, jnp.int32)
    zero_cnt = jnp.sum((bits == 0).astype(jnp.int32), axis=1, keepdims=True)
    mn = jnp.min(jnp.where(bits == 0, MAX_FINITE_BITS, bits),
                 axis=1, keepdims=True)
    mx = jnp.max(bits, axis=1, keepdims=True)
    lo = jnp.where(zero_cnt >= KNN, 0, jnp.minimum(mn, mx))
    hi = mx

    def bs_cond(lohi):
        return jnp.any(lohi[0] < lohi[1])

    def bs(lohi):
        lo, hi = lohi
        mid = lo + lax.div(hi - lo, 2)
        cnt = jnp.sum((bits <= mid).astype(jnp.int32), axis=1, keepdims=True)
        ge = cnt >= KNN
        return jnp.where(ge, lo, mid + 1), jnp.where(ge, mid, hi)

    _, tau_bits = lax.while_loop(bs_cond, bs, (lo, hi))
    tau2 = lax.bitcast_convert_type(tau_bits, jnp.float32)
    lt = bits < tau_bits
    c1 = jnp.sum(lt.astype(jnp.float32), axis=1, keepdims=True)
    ed = jnp.exp(-jnp.sqrt(d2) * INV_SCALE)
    s1 = jnp.sum(jnp.where(lt, ed, 0.0), axis=1, keepdims=True)
    te = jnp.exp(-jnp.sqrt(tau2) * INV_SCALE)
    dens_ref[...] = s1 + (jnp.float32(KNN) - c1) * te
    tau_ref[...] = tau2


def _rank_body(densc_ref, densr_ref, rank_ref):
    i = pl.program_id(0)
    du = densc_ref[...]
    dw = densr_ref[...]
    iu = lax.broadcasted_iota(jnp.int32, (RB, 1), 0) + i * RB
    iw = lax.broadcasted_iota(jnp.int32, (1, N), 1)
    less = (dw < du) | ((dw == du) & (iw < iu))
    rank_ref[...] = jnp.sum(less.astype(jnp.int32), axis=1, keepdims=True)


def _cc_body(x_ref, xt_ref, n2c_ref, n2r_ref, tauc_ref, taur_ref,
             rankc_ref, rankr_ref, pc_ref, pr_ref, pout_ref):
    d2 = _dist2(x_ref[...], xt_ref[...], n2c_ref[...], n2r_ref[...])
    ru = rankc_ref[...]
    rv = rankr_ref[...]
    pv = pr_ref[...]
    taueff = jnp.where(rv > ru, tauc_ref[...],
                       jnp.where(rv < ru, taur_ref[...], -1.0))
    edge = d2 <= taueff
    pnew = jnp.maximum(
        pc_ref[...],
        jnp.max(jnp.where(edge, pv, -1), axis=1, keepdims=True))
    for _ in range(2):
        pnew = jnp.max(jnp.where(rv == pnew, pv, -1), axis=1, keepdims=True)
    pout_ref[...] = pnew


def _label_body(pc_ref, pr_ref, rankr_ref, lab_ref):
    pu = pc_ref[...]
    pw = pr_ref[...]
    rw = rankr_ref[...]
    isroot = pw == rw
    lab_ref[...] = jnp.sum((isroot & (rw < pu)).astype(jnp.int32),
                           axis=1, keepdims=True)


def kernel(x):
    x = x.astype(jnp.float32)
    xt = x.T

    f32 = jnp.float32
    i32 = jnp.int32
    sds = jax.ShapeDtypeStruct

    n2 = pl.pallas_call(
        _norms_body, out_shape=sds((N, 1), f32))(x)
    n2r = n2.reshape(1, N)

    spec_x = pl.BlockSpec((RB, D), lambda i: (i, 0))
    spec_xt = pl.BlockSpec((D, N), lambda i: (0, 0))
    spec_col = pl.BlockSpec((RB, 1), lambda i: (i, 0))
    spec_row = pl.BlockSpec((1, N), lambda i: (0, 0))

    dens, tau = pl.pallas_call(
        _select_body,
        grid=(NBLK,),
        in_specs=[spec_x, spec_xt, spec_col, spec_row],
        out_specs=[spec_col, spec_col],
        out_shape=[sds((N, 1), f32), sds((N, 1), f32)],
    )(x, xt, n2, n2r)
    densr = dens.reshape(1, N)
    taur = tau.reshape(1, N)

    rank = pl.pallas_call(
        _rank_body,
        grid=(NBLK,),
        in_specs=[spec_col, spec_row],
        out_specs=spec_col,
        out_shape=sds((N, 1), i32),
    )(dens, densr)
    rankr = rank.reshape(1, N)

    cc_call = pl.pallas_call(
        _cc_body,
        grid=(NBLK,),
        in_specs=[spec_x, spec_xt, spec_col, spec_row, spec_col, spec_row,
                  spec_col, spec_row, spec_col, spec_row],
        out_specs=spec_col,
        out_shape=sds((N, 1), i32),
    )

    def cond(carry):
        return carry[1]

    def body(carry):
        p, _ = carry
        pn = cc_call(x, xt, n2, n2r, tau, taur, rank, rankr,
                     p, p.reshape(1, N))
        return pn, jnp.any(pn != p)

    p, _ = lax.while_loop(cond, body, (rank, jnp.bool_(True)))

    labels = pl.pallas_call(
        _label_body,
        grid=(NBLK,),
        in_specs=[spec_col, spec_row, spec_row],
        out_specs=spec_col,
        out_shape=sds((N, 1), i32),
    )(p, p.reshape(1, N), rankr)

    return jnp.asarray(labels.reshape(N), jnp.int64)


# speedup vs baseline: 1.3721x; 1.0777x over previous
import jax
import jax.numpy as jnp
from jax import lax
from jax.experimental import pallas as pl

N = 8192
D = 16
KNN = 64
INV_SCALE = 2.0
RB = 512
NBLK = N // RB
MAX_FINITE_BITS = 0x7F7FFFFF


def _dist2(xb, xt, n2c, n2r):
    g = lax.dot_general(xb, xt, (((1,), (0,)), ((), ())),
                        preferred_element_type=jnp.float32)
    return jnp.maximum(n2c + n2r - 2.0 * g, 0.0)


def _norms_body(x_ref, n2_ref):
    xb = x_ref[...]
    n2_ref[...] = jnp.sum(xb * xb, axis=1, keepdims=True)


def _select_body(x_ref, xt_ref, n2c_ref, n2r_ref, dens_ref, tau_ref):
    d2 = _dist2(x_ref[...], xt_ref[...], n2c_ref[...], n2r_ref[...])
    bits = lax.bitcast_convert_type(d2, jnp.int32)
    zero_cnt = jnp.sum((bits == 0).astype(jnp.int32), axis=1, keepdims=True)
    mn = jnp.min(jnp.where(bits == 0, MAX_FINITE_BITS, bits),
                 axis=1, keepdims=True)
    mx = jnp.max(bits, axis=1, keepdims=True)
    lo = jnp.where(zero_cnt >= KNN, 0, mn)
    hi = jnp.maximum(mx, lo)

    def bs_cond(lohi):
        return jnp.any(lohi[0] < lohi[1])

    def bs(lohi):
        lo, hi = lohi
        mid = lo + lax.div(hi - lo, 2)
        cnt = jnp.sum((bits <= mid).astype(jnp.int32), axis=1, keepdims=True)
        ge = cnt >= KNN
        return jnp.where(ge, lo, mid + 1), jnp.where(ge, mid, hi)

    _, tau_bits = lax.while_loop(bs_cond, bs, (lo, hi))
    tau2 = lax.bitcast_convert_type(tau_bits, jnp.float32)
    lt = bits < tau_bits
    c1 = jnp.sum(lt.astype(jnp.float32), axis=1, keepdims=True)
    ed = jnp.exp(-jnp.sqrt(d2) * INV_SCALE)
    s1 = jnp.sum(jnp.where(lt, ed, 0.0), axis=1, keepdims=True)
    te = jnp.exp(-jnp.sqrt(tau2) * INV_SCALE)
    dens_ref[...] = s1 + (jnp.float32(KNN) - c1) * te
    tau_ref[...] = tau2


def _rank_body(densc_ref, densr_ref, rank_ref):
    i = pl.program_id(0)
    du = densc_ref[...]
    dw = densr_ref[...]
    iu = lax.broadcasted_iota(jnp.int32, (RB, 1), 0) + i * RB
    iw = lax.broadcasted_iota(jnp.int32, (1, N), 1)
    less = (dw < du) | ((dw == du) & (iw < iu))
    rank_ref[...] = jnp.sum(less.astype(jnp.int32), axis=1, keepdims=True)


def _cc_body(x_ref, xt_ref, n2c_ref, n2r_ref, tauc_ref, taur_ref,
             rankc_ref, rankr_ref, pc_ref, pr_ref, pout_ref):
    d2 = _dist2(x_ref[...], xt_ref[...], n2c_ref[...], n2r_ref[...])
    ru = rankc_ref[...]
    rv = rankr_ref[...]
    pv = pr_ref[...]
    taueff = jnp.where(rv > ru, tauc_ref[...],
                       jnp.where(rv < ru, taur_ref[...], -1.0))
    edge = d2 <= taueff
    pnew = jnp.maximum(
        pc_ref[...],
        jnp.max(jnp.where(edge, pv, -1), axis=1, keepdims=True))
    pnew = jnp.max(jnp.where(rv == pnew, pv, -1), axis=1, keepdims=True)
    pout_ref[...] = pnew


def _label_body(pc_ref, pr_ref, rankr_ref, lab_ref):
    pu = pc_ref[...]
    pw = pr_ref[...]
    rw = rankr_ref[...]
    isroot = pw == rw
    lab_ref[...] = jnp.sum((isroot & (rw < pu)).astype(jnp.int32),
                           axis=1, keepdims=True)


def kernel(x):
    x = x.astype(jnp.float32)
    xt = x.T

    f32 = jnp.float32
    i32 = jnp.int32
    sds = jax.ShapeDtypeStruct

    n2 = pl.pallas_call(
        _norms_body, out_shape=sds((N, 1), f32))(x)
    n2r = n2.reshape(1, N)

    spec_x = pl.BlockSpec((RB, D), lambda i: (i, 0))
    spec_xt = pl.BlockSpec((D, N), lambda i: (0, 0))
    spec_col = pl.BlockSpec((RB, 1), lambda i: (i, 0))
    spec_row = pl.BlockSpec((1, N), lambda i: (0, 0))

    dens, tau = pl.pallas_call(
        _select_body,
        grid=(NBLK,),
        in_specs=[spec_x, spec_xt, spec_col, spec_row],
        out_specs=[spec_col, spec_col],
        out_shape=[sds((N, 1), f32), sds((N, 1), f32)],
    )(x, xt, n2, n2r)
    densr = dens.reshape(1, N)
    taur = tau.reshape(1, N)

    rank = pl.pallas_call(
        _rank_body,
        grid=(NBLK,),
        in_specs=[spec_col, spec_row],
        out_specs=spec_col,
        out_shape=sds((N, 1), i32),
    )(dens, densr)
    rankr = rank.reshape(1, N)

    cc_call = pl.pallas_call(
        _cc_body,
        grid=(NBLK,),
        in_specs=[spec_x, spec_xt, spec_col, spec_row, spec_col, spec_row,
                  spec_col, spec_row, spec_col, spec_row],
        out_specs=spec_col,
        out_shape=sds((N, 1), i32),
    )

    def cond(carry):
        return carry[1]

    def body(carry):
        p, _ = carry
        pn = cc_call(x, xt, n2, n2r, tau, taur, rank, rankr,
                     p, p.reshape(1, N))
        return pn, jnp.any(pn != p)

    p, _ = lax.while_loop(cond, body, (rank, jnp.bool_(True)))

    labels = pl.pallas_call(
        _label_body,
        grid=(NBLK,),
        in_specs=[spec_col, spec_row, spec_row],
        out_specs=spec_col,
        out_shape=sds((N, 1), i32),
    )(p, p.reshape(1, N), rankr)

    return jnp.asarray(labels.reshape(N), jnp.int64)
